# baseline (speedup 1.0000x reference)
import jax
import jax.numpy as jnp
from jax import lax
from jax.experimental import pallas as pl
from jax.experimental.pallas import tpu as pltpu, tpu_sc as plsc

_B = 16384
_D = 64
_NC = 2
_NS = 16
_NW = _NC * _NS
_BPW = _B // _NW
_CHUNK = 128
_NCHUNK = _BPW // _CHUNK
_NGATHER = 6
_NTASK = _NGATHER * _NCHUNK
_NBUF = 4

_mesh = plsc.VectorSubcoreMesh(
    core_axis_name="c", subcore_axis_name="s",
    num_cores=_NC, num_subcores=_NS,
)


def _body(user_pairs, track_pairs, p0, p1, p2, p3, p4, p5,
          o0, o1, o2, o3, o4, o5,
          u_out, tp_out, tn_out, up_out, un_out, ta_out,
          pidx_v, poff_v, pairbuf, cols_v, sem):
    wid = lax.axis_index("s") * _NC + lax.axis_index("c")
    base = wid * _BPW

    tables = (user_pairs, track_pairs, track_pairs,
              user_pairs, user_pairs, track_pairs)
    pair_in = (p0, p1, p2, p3, p4, p5)
    off_in = (o0, o1, o2, o3, o4, o5)
    outs = (u_out, tp_out, tn_out, up_out, un_out, ta_out)

    for g in range(_NGATHER):
        pltpu.sync_copy(pair_in[g].at[wid], pidx_v.at[g])
        pltpu.sync_copy(off_in[g].at[wid], poff_v.at[g])

    lanes = lax.iota(jnp.int32, 16)

    def fire(t):
        g, c = divmod(t, _NCHUNK)
        pltpu.async_copy(
            tables[g].at[pidx_v.at[g, c]], pairbuf.at[t % _NBUF], sem)

    def wait_one():
        pltpu.make_async_copy(
            tables[0].at[pidx_v.at[0, 0]], pairbuf.at[0], sem).wait()

    for t in range(_NBUF):
        fire(t)
    for t in range(_NTASK):
        g, c = divmod(t, _NCHUNK)
        buf = pairbuf.at[t % _NBUF]
        wait_one()

        offv = [poff_v[g, c, pl.ds(jb * 16, 16)] for jb in range(8)]

        def do_dim(d):
            for jb in range(8):
                v = plsc.load_gather(buf, [jb * 16 + lanes, offv[jb] + d])
                cols_v[d, pl.ds(jb * 16, 16)] = v

        pl.loop(0, _D)(do_dim)
        pltpu.sync_copy(
            cols_v, outs[g].at[:, pl.ds(base + c * _CHUNK, _CHUNK)])
        if t + _NBUF < _NTASK:
            fire(t + _NBUF)


_out_struct = jax.ShapeDtypeStruct((_D, _B), jnp.float32)

_gather6 = pl.kernel(
    _body,
    out_type=(_out_struct,) * _NGATHER,
    mesh=_mesh,
    scratch_types=(
        pltpu.VMEM((_NGATHER, _NCHUNK, _CHUNK), jnp.int32),
        pltpu.VMEM((_NGATHER, _NCHUNK, _CHUNK), jnp.int32),
        pltpu.VMEM((_NBUF, _CHUNK, 2 * _D), jnp.float32),
        pltpu.VMEM((_D, _CHUNK), jnp.float32),
        pltpu.SemaphoreType.DMA,
    ),
    compiler_params=pltpu.CompilerParams(
        use_tc_tiling_on_sc=True, needs_layout_passes=False),
)


def kernel(user_mat, track_mat, x_user, x_track_pos, x_track_neg,
           x_user_pos, x_user_neg, x_track_anchor):
    up = jnp.concatenate([user_mat[0::2], user_mat[1::2]], axis=1)
    tp = jnp.concatenate([track_mat[0::2], track_mat[1::2]], axis=1)

    def prep(x):
        x = x.astype(jnp.int32)
        pair = jnp.reshape(x >> 1, (_NW, _NCHUNK, _CHUNK))
        off = jnp.reshape((x & 1) << 6, (_NW, _NCHUNK, _CHUNK))
        return pair, off

    pr = [prep(x) for x in (x_user, x_track_pos, x_track_neg,
                            x_user_pos, x_user_neg, x_track_anchor)]
    outs = _gather6(
        up, tp,
        pr[0][0], pr[1][0], pr[2][0], pr[3][0], pr[4][0], pr[5][0],
        pr[0][1], pr[1][1], pr[2][1], pr[3][1], pr[4][1], pr[5][1],
    )
    return tuple(o.T for o in outs)


# speedup vs baseline: 20.5971x; 20.5971x over previous
import jax
import jax.numpy as jnp
from jax import lax
from jax.experimental import pallas as pl
from jax.experimental.pallas import tpu as pltpu, tpu_sc as plsc

_B = 16384
_D = 64
_NC = 2
_NS = 16
_NW = _NC * _NS
_BPW = _B // _NW
_NV = _BPW // 16
_NGATHER = 3
_RING = 32

_mesh = plsc.VectorSubcoreMesh(
    core_axis_name="c", subcore_axis_name="s",
    num_cores=_NC, num_subcores=_NS,
)


def _body(tab_hbm, x0, x1, x2, out0, out1, out2, idx_v, ring, cols_v, sem):
    wid = lax.axis_index("s") * _NC + lax.axis_index("c")
    base = wid * _BPW

    idx_in = (x0, x1, x2)
    outs = (out0, out1, out2)

    for g in range(_NGATHER):
        pltpu.sync_copy(idx_in[g].at[wid], idx_v.at[g])

    lane = lax.iota(jnp.int32, 16)

    def extract(vec, k):
        return lax.reduce_max(jnp.where(lane == k, vec, 0), axes=(0,))

    for g in range(_NGATHER):
        gv = idx_v.at[g]

        def fire(v):
            vec = gv[v, :]
            iks = tuple(extract(vec, k) for k in range(16))
            for k in range(16):
                off = pl.multiple_of(iks[k] & ~jnp.int32(7), 8)
                pltpu.async_copy(
                    tab_hbm.at[pl.ds(off, 8), :],
                    ring.at[(v * 16 + k) % _RING], sem)
            return iks

        def drain_and_transpose(v, iks):
            for k in range(16):
                pltpu.make_async_copy(
                    tab_hbm.at[pl.ds(0, 8), :], ring.at[0], sem).wait()
            j0 = v * 16
            for k in range(16):
                i7 = iks[k] & 7
                slot = (j0 + k) % _RING
                for b in range(_D // 16):
                    vv = ring[slot, i7, pl.ds(b * 16, 16)]
                    plsc.store_scatter(
                        cols_v,
                        [b * 16 + lane, jnp.full((16,), j0 + k, jnp.int32)],
                        vv)

        sub0 = fire(0)

        def step(v, iks):
            nxt = lax.cond(v + 1 < _NV, lambda: fire(v + 1), lambda: iks)
            drain_and_transpose(v, iks)
            return nxt

        pl.loop(0, _NV, init_carry=sub0)(step)
        pltpu.sync_copy(cols_v, outs[g].at[:, pl.ds(base, _BPW)])


_out_struct = jax.ShapeDtypeStruct((_D, _B), jnp.float32)

_gather3 = pl.kernel(
    _body,
    out_type=(_out_struct,) * _NGATHER,
    mesh=_mesh,
    scratch_types=(
        pltpu.VMEM((_NGATHER, _NV, 16), jnp.int32),
        pltpu.VMEM((_RING, 8, _D), jnp.float32),
        pltpu.VMEM((_D, _BPW), jnp.float32),
        pltpu.SemaphoreType.DMA,
    ),
    compiler_params=pltpu.CompilerParams(
        use_tc_tiling_on_sc=True, needs_layout_passes=False),
)


def kernel(user_mat, track_mat, x_user, x_track_pos, x_track_neg,
           x_user_pos, x_user_neg, x_track_anchor):
    def prep(x):
        return jnp.reshape(x.astype(jnp.int32), (_NW, _NV, 16))

    u, u_pos, u_neg = _gather3(
        user_mat, prep(x_user), prep(x_user_pos), prep(x_user_neg))
    t_pos, t_neg, t_anchor = _gather3(
        track_mat, prep(x_track_pos), prep(x_track_neg),
        prep(x_track_anchor))
    return (u.T, t_pos.T, t_neg.T, u_pos.T, u_neg.T, t_anchor.T)
